# baseline (speedup 1.0000x reference)
import functools

import jax
import jax.numpy as jnp
from jax import lax
from jax.experimental import pallas as pl
from jax.experimental.pallas import tpu as pltpu
from jax.experimental.pallas import tpu_sc as plsc

B, K, N = 1024, 1024, 256

_NC, _NS = 2, 16
_NW = _NC * _NS
_BPW = B // _NW


_BB = 512


def _argmax_body(hd_ref, c_ref, idx_ref):
    x = (2.0 * K) * hd_ref[:, 0, :] - float(K)
    t = lax.dot_general(
        x, c_ref[...],
        dimension_numbers=(((1,), (1,)), ((), ())),
        preferred_element_type=jnp.float32,
    )
    rev_k = lax.broadcasted_iota(jnp.int32, (1, K), 1).astype(jnp.float32)
    s = t + (float(K - 1) - rev_k)
    m = jnp.max(s, axis=1)
    r = m - float(K) * jnp.floor(m * (1.0 / K))
    idx_ref[...] = (K - 1) - r.astype(jnp.int32)


def _tc_argmax(hd3, C):
    return pl.pallas_call(
        _argmax_body,
        grid=(B // _BB,),
        in_specs=[
            pl.BlockSpec((_BB, 1, N), lambda i: (i, 0, 0)),
            pl.BlockSpec((K, N), lambda i: (0, 0)),
        ],
        out_specs=pl.BlockSpec((_BB,), lambda i: (i,)),
        out_shape=jax.ShapeDtypeStruct((B,), jnp.int32),
    )(hd3, C)


@functools.lru_cache(maxsize=4)
def _make_sc_gather(rows):
    bpw = rows // _NW
    nch = 4
    ch = bpw // nch

    @functools.partial(
        pl.kernel,
        out_type=jax.ShapeDtypeStruct((rows, N), jnp.float32),
        mesh=plsc.VectorSubcoreMesh(core_axis_name="c", subcore_axis_name="s"),
        scratch_types=[
            pltpu.VMEM((bpw,), jnp.int32),
            pltpu.VMEM((4, bpw // 4, N), jnp.float32),
            pltpu.SemaphoreType.DMA,
            pltpu.SemaphoreType.DMA,
        ],
    )
    def _sc_gather(table_hbm, idx_hbm, out_hbm, idx_v, rows_v, gsem, wsem):
        wid = lax.axis_index("s") * _NC + lax.axis_index("c")
        base = wid * bpw
        pltpu.sync_copy(idx_hbm.at[pl.ds(base, bpw)], idx_v)
        gathers = [
            pltpu.async_copy(
                table_hbm.at[idx_v.at[pl.ds(j * ch, ch)]], rows_v.at[j], gsem)
            for j in range(nch)
        ]
        writes = []
        for j in range(nch):
            gathers[j].wait()
            writes.append(pltpu.async_copy(
                rows_v.at[j], out_hbm.at[pl.ds(base + j * ch, ch)], wsem))
        for w in writes:
            w.wait()

    return _sc_gather


def kernel(harddecision, C):
    idx = _tc_argmax(harddecision, C)
    out = _make_sc_gather(B)(C, idx)
    return out[:, None, :]


# speedup vs baseline: 1.0032x; 1.0032x over previous
import functools

import jax
import jax.numpy as jnp
from jax import lax
from jax.experimental import pallas as pl
from jax.experimental.pallas import tpu as pltpu
from jax.experimental.pallas import tpu_sc as plsc

B, K, N = 1024, 1024, 256

_NC, _NS = 2, 16
_NW = _NC * _NS
_BPW = B // _NW


_BB = 512


def _argmax_body(hd_ref, c_ref, idx_ref):
    x = (2.0 * K) * hd_ref[:, 0, :] - float(K)
    t = lax.dot_general(
        x, c_ref[...],
        dimension_numbers=(((1,), (1,)), ((), ())),
        preferred_element_type=jnp.float32,
    )
    rev_k = lax.broadcasted_iota(jnp.int32, (1, K), 1).astype(jnp.float32)
    s = t + (float(K - 1) - rev_k)
    m = jnp.max(s, axis=1)
    r = m - float(K) * jnp.floor(m * (1.0 / K))
    idx_ref[...] = (K - 1) - r.astype(jnp.int32)


def _tc_argmax(hd3, C):
    return pl.pallas_call(
        _argmax_body,
        grid=(B // _BB,),
        in_specs=[
            pl.BlockSpec((_BB, 1, N), lambda i: (i, 0, 0)),
            pl.BlockSpec((K, N), lambda i: (0, 0)),
        ],
        out_specs=pl.BlockSpec((_BB,), lambda i: (i,)),
        out_shape=jax.ShapeDtypeStruct((B,), jnp.int32),
    )(hd3, C)


@functools.lru_cache(maxsize=4)
def _make_sc_gather(rows):
    bpw = rows // _NW
    nch = 2
    ch = bpw // nch

    @functools.partial(
        pl.kernel,
        out_type=jax.ShapeDtypeStruct((rows, N), jnp.float32),
        mesh=plsc.VectorSubcoreMesh(core_axis_name="c", subcore_axis_name="s"),
        scratch_types=[
            pltpu.VMEM((bpw,), jnp.int32),
            pltpu.VMEM((nch, bpw // nch, N), jnp.float32),
            pltpu.SemaphoreType.DMA,
            pltpu.SemaphoreType.DMA,
        ],
    )
    def _sc_gather(table_hbm, idx_hbm, out_hbm, idx_v, rows_v, gsem, wsem):
        wid = lax.axis_index("s") * _NC + lax.axis_index("c")
        base = wid * bpw
        pltpu.sync_copy(idx_hbm.at[pl.ds(base, bpw)], idx_v)
        gathers = [
            pltpu.async_copy(
                table_hbm.at[idx_v.at[pl.ds(j * ch, ch)]], rows_v.at[j], gsem)
            for j in range(nch)
        ]
        writes = []
        for j in range(nch):
            gathers[j].wait()
            writes.append(pltpu.async_copy(
                rows_v.at[j], out_hbm.at[pl.ds(base + j * ch, ch)], wsem))
        for w in writes:
            w.wait()

    return _sc_gather


def kernel(harddecision, C):
    idx = _tc_argmax(harddecision, C)
    out = _make_sc_gather(B)(C, idx)
    return out[:, None, :]
